# baseline (speedup 1.0000x reference)
import functools

import jax
import jax.numpy as jnp
from jax import lax
from jax.experimental import pallas as pl
from jax.experimental.pallas import tpu as pltpu
from jax.experimental.pallas import tpu_sc as plsc

EMBED = 64
NC = 2
NS = 16
NW = NC * NS

CHUNK = 128
LANES = 16
NBUF = 2
TPAIR = 513


def _make_sc_gather(B):
    PW = B // NW
    G = PW // CHUNK

    mesh = plsc.VectorSubcoreMesh(core_axis_name="c", subcore_axis_name="s")

    @functools.partial(
        pl.kernel,
        mesh=mesh,
        out_type=jax.ShapeDtypeStruct((B, EMBED), jnp.float32),
        scratch_types=[
            pltpu.VMEM((1, CHUNK), jnp.int32),
            pltpu.VMEM((1, CHUNK), jnp.int32),
            pltpu.VMEM((1, CHUNK), jnp.int32),
            pltpu.VMEM((1, CHUNK), jnp.int32),
            pltpu.VMEM((CHUNK, 2 * EMBED), jnp.float32),
            pltpu.VMEM((CHUNK, 2 * EMBED), jnp.float32),
            pltpu.VMEM((CHUNK, EMBED), jnp.float32),
            pltpu.VMEM((CHUNK, EMBED), jnp.float32),
            pltpu.VMEM_SHARED((TPAIR, 2 * EMBED), jnp.float32),
            pltpu.SemaphoreType.DMA,
            pltpu.SemaphoreType.DMA,
            pltpu.SemaphoreType.DMA,
            pltpu.SemaphoreType.DMA,
            pltpu.SemaphoreType.DMA,
            pltpu.SemaphoreType.DMA,
        ],
    )
    def k(idx_hbm, table_hbm, out_hbm,
          idx_v0, idx_v1, idx2_v0, idx2_v1,
          rows_v0, rows_v1, rt_v0, rt_v1, tab_sh,
          si0, si1, sg0, sg1, so0, so1):
        wid = lax.axis_index("s") * NC + lax.axis_index("c")
        idx_v = (idx_v0, idx_v1)
        idx2_v = (idx2_v0, idx2_v1)
        rows_v = (rows_v0, rows_v1)
        rt_v = (rt_v0, rt_v1)
        sem_i = (si0, si1)
        sem_g = (sg0, sg1)
        sem_o = (so0, so1)

        pltpu.sync_copy(table_hbm, tab_sh)

        def fetch_idx(slot, c):
            pltpu.async_copy(idx_hbm.at[wid, c], idx_v[slot], sem_i[slot])

        def wait_idx(slot, c):
            pltpu.make_async_copy(
                idx_hbm.at[wid, c], idx_v[slot], sem_i[slot]).wait()

        def fire_gather(slot):
            for gsl in range(CHUNK // LANES):
                v = idx_v[slot][0, pl.ds(gsl * LANES, LANES)]
                idx2_v[slot][0, pl.ds(gsl * LANES, LANES)] = (
                    lax.shift_right_logical(v, 1))
            pltpu.async_copy(
                tab_sh.at[idx2_v[slot].at[0]], rows_v[slot], sem_g[slot])

        def wait_gather(slot):
            pltpu.make_async_copy(
                tab_sh.at[idx2_v[slot].at[0]], rows_v[slot], sem_g[slot]
            ).wait()

        def repack_and_flush(slot, p):
            def repack(g, carry):
                par = (idx_v[slot][0, pl.ds(g * LANES, LANES)] & 1) * EMBED
                for dr in range(LANES):
                    r = g * LANES + dr
                    off = par[dr]
                    for cc in range(EMBED // LANES):
                        rt_v[slot][r, pl.ds(cc * LANES, LANES)] = (
                            rows_v[slot][r, pl.ds(off + cc * LANES, LANES)])
                return carry

            lax.fori_loop(0, CHUNK // LANES, repack, 0)
            base = wid * PW + p * CHUNK
            pltpu.async_copy(
                rt_v[slot], out_hbm.at[pl.ds(base, CHUNK)], sem_o[slot])

        def wait_flush(slot):
            base = wid * PW
            pltpu.make_async_copy(
                rt_v[slot], out_hbm.at[pl.ds(base, CHUNK)], sem_o[slot]
            ).wait()

        for b in range(NBUF):
            fetch_idx(b, b)

        def body(c2, carry):
            for b in range(NBUF):
                c = NBUF * c2 + b
                q = 1 - b
                wait_idx(b, c)
                fire_gather(b)

                def handle_prev():
                    wait_gather(q)
                    repack_and_flush(q, c - 1)

                    @pl.when(c + 1 < G)
                    def _():
                        fetch_idx(q, c + 1)

                if b == 0:
                    @pl.when(c2 >= 1)
                    def _():
                        @pl.when(c2 >= 2)
                        def _():
                            wait_flush(q)
                        handle_prev()
                else:
                    @pl.when(c2 >= 1)
                    def _():
                        wait_flush(q)
                    handle_prev()
            return carry

        lax.fori_loop(0, G // NBUF, body, 0)

        qf = (G - 1) % NBUF
        wait_flush(qf)
        wait_gather(qf)
        repack_and_flush(qf, G - 1)
        wait_flush(1 - qf)
        wait_flush(qf)

    return k


def kernel(positions, table):
    batch, seq = positions.shape
    B = batch * seq
    idx = positions.reshape(NW, B // (NW * CHUNK), 1, CHUNK).astype(jnp.int32)
    table_pair = jnp.pad(table, ((0, 1), (0, 0))).reshape(TPAIR, 2 * EMBED)
    out = _make_sc_gather(B)(idx, table_pair)
    return out.reshape(batch, seq, EMBED)


# speedup vs baseline: 1.0963x; 1.0963x over previous
import functools

import jax
import jax.numpy as jnp
from jax import lax
from jax.experimental import pallas as pl
from jax.experimental.pallas import tpu as pltpu
from jax.experimental.pallas import tpu_sc as plsc

EMBED = 64
NC = 2
NS = 16
NW = NC * NS

CHUNK = 128
LANES = 16
NBUF = 2


def _make_sc_gather(B):
    PW = B // NW
    G = PW // CHUNK

    mesh = plsc.VectorSubcoreMesh(core_axis_name="c", subcore_axis_name="s")

    @functools.partial(
        pl.kernel,
        mesh=mesh,
        out_type=jax.ShapeDtypeStruct((B, EMBED), jnp.float32),
        scratch_types=[
            pltpu.VMEM((1, CHUNK), jnp.int32),
            pltpu.VMEM((1, CHUNK), jnp.int32),
            pltpu.VMEM((CHUNK, 2 * EMBED), jnp.float32),
            pltpu.VMEM((CHUNK, 2 * EMBED), jnp.float32),
            pltpu.VMEM((CHUNK, EMBED), jnp.float32),
            pltpu.VMEM((CHUNK, EMBED), jnp.float32),
            pltpu.SemaphoreType.DMA,
            pltpu.SemaphoreType.DMA,
            pltpu.SemaphoreType.DMA,
            pltpu.SemaphoreType.DMA,
            pltpu.SemaphoreType.DMA,
            pltpu.SemaphoreType.DMA,
        ],
    )
    def k(idx_hbm, table_hbm, out_hbm,
          idx_v0, idx_v1, rows_v0, rows_v1, rt_v0, rt_v1,
          si0, si1, sg0, sg1, so0, so1):
        wid = lax.axis_index("s") * NC + lax.axis_index("c")
        idx_v = (idx_v0, idx_v1)
        rows_v = (rows_v0, rows_v1)
        rt_v = (rt_v0, rt_v1)
        sem_i = (si0, si1)
        sem_g = (sg0, sg1)
        sem_o = (so0, so1)

        def fire_gather(slot, c):
            pltpu.async_copy(
                table_hbm.at[idx_v[slot].at[0]], rows_v[slot], sem_g[slot])

        def wait_gather(slot):
            pltpu.make_async_copy(
                table_hbm.at[idx_v[slot].at[0]], rows_v[slot], sem_g[slot]
            ).wait()

        def repack_and_flush(slot, p):
            def repack(r4, carry):
                for dr in range(4):
                    r = r4 * 4 + dr
                    for cc in range(EMBED // LANES):
                        rt_v[slot][r, pl.ds(cc * LANES, LANES)] = (
                            rows_v[slot][r, pl.ds(cc * LANES, LANES)])
                return carry

            lax.fori_loop(0, CHUNK // 4, repack, 0)
            base = wid * PW + p * CHUNK
            pltpu.async_copy(
                rt_v[slot], out_hbm.at[pl.ds(base, CHUNK)], sem_o[slot])

        def wait_flush(slot):
            base = wid * PW
            pltpu.make_async_copy(
                rt_v[slot], out_hbm.at[pl.ds(base, CHUNK)], sem_o[slot]
            ).wait()

        for b in range(NBUF):
            pltpu.async_copy(idx_hbm.at[wid, b], idx_v[b], sem_i[b])

        def body(c2, carry):
            for b in range(NBUF):
                c = NBUF * c2 + b
                q = 1 - b
                pltpu.make_async_copy(
                    idx_hbm.at[wid, c], idx_v[b], sem_i[b]).wait()
                fire_gather(b, c)

                def handle_prev():
                    wait_gather(q)
                    repack_and_flush(q, c - 1)

                    @pl.when(c + 1 < G)
                    def _():
                        pltpu.async_copy(
                            idx_hbm.at[wid, c + 1], idx_v[q], sem_i[q])

                if b == 0:
                    @pl.when(c2 >= 1)
                    def _():
                        @pl.when(c2 >= 2)
                        def _():
                            wait_flush(q)
                        handle_prev()
                else:
                    @pl.when(c2 >= 1)
                    def _():
                        wait_flush(q)
                    handle_prev()
            return carry

        lax.fori_loop(0, G // NBUF, body, 0)

        qf = (G - 1) % NBUF
        wait_flush(qf)
        wait_gather(qf)
        repack_and_flush(qf, G - 1)
        wait_flush(1 - qf)
        wait_flush(qf)

    return k


def kernel(positions, table):
    batch, seq = positions.shape
    B = batch * seq
    idx = positions.reshape(NW, B // (NW * CHUNK), 1, CHUNK).astype(jnp.int32)
    table_pad = jnp.pad(table, ((0, 0), (0, EMBED)))
    out = _make_sc_gather(B)(idx, table_pad)
    return out.reshape(batch, seq, EMBED)
